# baseline (speedup 1.0000x reference)
import jax
import jax.numpy as jnp
from jax.experimental import pallas as pl
from jax.experimental.pallas import tpu as pltpu

_BB = 8


def _broadcast_dma_body(w_ref, o_ref, buf_ref, sem):
    buf_ref[...] = jnp.broadcast_to(w_ref[...][None], buf_ref.shape)
    n = o_ref.shape[0] // _BB
    copies = [
        pltpu.make_async_copy(buf_ref, o_ref.at[pl.ds(i * _BB, _BB)], sem)
        for i in range(n)
    ]
    for c in copies:
        c.start()
    for c in copies:
        c.wait()


def kernel(x, W):
    B, P, D = x.shape
    out = pl.pallas_call(
        _broadcast_dma_body,
        in_specs=[pl.BlockSpec(memory_space=pltpu.MemorySpace.VMEM)],
        out_specs=pl.BlockSpec(memory_space=pltpu.MemorySpace.HBM),
        out_shape=jax.ShapeDtypeStruct((B, P, D), W.dtype),
        scratch_shapes=[
            pltpu.VMEM((_BB, P, D), W.dtype),
            pltpu.SemaphoreType.DMA,
        ],
    )(W)
    return out
